# baseline (speedup 1.0000x reference)
import jax
import jax.numpy as jnp
from jax.experimental import pallas as pl
from jax.experimental.pallas import tpu as pltpu

_TOP_K = 2
_BLOCK_T = 2048


def _router_block(x_ref, wt_ref, idx_ref, w_ref):
    x = x_ref[...]
    wt = wt_ref[...]
    logits = jnp.dot(x, wt, preferred_element_type=jnp.float32)
    lt = logits.T
    n_e = lt.shape[0]
    rows = [lt[e] for e in range(n_e)]

    m1 = rows[0]
    for e in range(1, n_e):
        m1 = jnp.maximum(m1, rows[e])
    i1 = jnp.full_like(m1, n_e - 1, dtype=jnp.int32)
    for e in range(n_e - 2, -1, -1):
        i1 = jnp.where(rows[e] == m1, e, i1)

    neg = jnp.float32(-3.0e38)
    rows2 = [jnp.where(i1 == e, neg, rows[e]) for e in range(n_e)]
    m2 = rows2[0]
    for e in range(1, n_e):
        m2 = jnp.maximum(m2, rows2[e])
    i2 = jnp.full_like(m1, n_e - 1, dtype=jnp.int32)
    for e in range(n_e - 2, -1, -1):
        i2 = jnp.where(rows2[e] == m2, e, i2)

    d = jnp.exp(m2 - m1)
    r = 1.0 / (1.0 + d)
    idx_ref[...] = jnp.stack([i1, i2], axis=-1)
    w_ref[...] = jnp.stack([r, d * r], axis=-1)


@jax.jit
def kernel(hidden_states, weight):
    bsz, seq_len, h = hidden_states.shape
    n_tok = bsz * seq_len
    n_exp = weight.shape[0]
    x = hidden_states.reshape(n_tok, h).astype(jnp.float32)
    wt = weight.astype(jnp.float32).T

    grid = (n_tok // _BLOCK_T,)
    topk_idx, topk_w = pl.pallas_call(
        _router_block,
        grid=grid,
        in_specs=[
            pl.BlockSpec((_BLOCK_T, h), lambda i: (i, 0)),
            pl.BlockSpec((h, n_exp), lambda i: (0, 0)),
        ],
        out_specs=[
            pl.BlockSpec((_BLOCK_T, _TOP_K), lambda i: (i, 0)),
            pl.BlockSpec((_BLOCK_T, _TOP_K), lambda i: (i, 0)),
        ],
        out_shape=[
            jax.ShapeDtypeStruct((n_tok, _TOP_K), jnp.int32),
            jax.ShapeDtypeStruct((n_tok, _TOP_K), jnp.float32),
        ],
        compiler_params=pltpu.CompilerParams(
            dimension_semantics=("arbitrary",),
        ),
    )(x, wt)
    return (topk_idx, topk_w)


# speedup vs baseline: 1.0814x; 1.0814x over previous
import functools

import jax
import jax.numpy as jnp
from jax import lax
from jax.experimental import pallas as pl
from jax.experimental.pallas import tpu as pltpu
from jax.experimental.pallas import tpu_sc as plsc

_TOP_K = 2
_BLOCK_T = 2048
_N_EXP = 8
_LANES = 16


def _logits_block(x_ref, wt_ref, lt_ref):
    x = x_ref[...]
    wt = wt_ref[...]
    logits = jnp.dot(x, wt, preferred_element_type=jnp.float32)
    lt_ref[...] = logits.T


def _tc_logits(x, wt):
    n_tok, h = x.shape
    n_exp = wt.shape[1]
    grid = (n_tok // _BLOCK_T,)
    return pl.pallas_call(
        _logits_block,
        grid=grid,
        in_specs=[
            pl.BlockSpec((_BLOCK_T, h), lambda i: (i, 0)),
            pl.BlockSpec((h, n_exp), lambda i: (0, 0)),
        ],
        out_specs=pl.BlockSpec((n_exp, _BLOCK_T), lambda i: (0, i)),
        out_shape=jax.ShapeDtypeStruct((n_exp, n_tok), jnp.float32),
        compiler_params=pltpu.CompilerParams(
            dimension_semantics=("arbitrary",),
        ),
    )(x, wt)


def _make_sc_route(n_tok):
    info = plsc.get_sparse_core_info()
    n_workers = info.num_cores * info.num_subcores
    chunk = n_tok // n_workers
    mesh = plsc.VectorSubcoreMesh(core_axis_name="c", subcore_axis_name="s")

    @functools.partial(
        pl.kernel,
        out_type=[
            jax.ShapeDtypeStruct((_TOP_K, n_tok), jnp.int32),
            jax.ShapeDtypeStruct((_TOP_K, n_tok), jnp.float32),
        ],
        mesh=mesh,
        scratch_types=[
            pltpu.VMEM((_N_EXP, chunk), jnp.float32),
            pltpu.VMEM((_TOP_K, chunk), jnp.int32),
            pltpu.VMEM((_TOP_K, chunk), jnp.float32),
        ],
    )
    def sc_route(lt_hbm, idx_hbm, w_hbm, rows_v, idx_v, w_v):
        wid = lax.axis_index("s") * info.num_cores + lax.axis_index("c")
        base = wid * chunk
        pltpu.sync_copy(lt_hbm.at[:, pl.ds(base, chunk)], rows_v)

        def body(j, carry):
            sl = pl.ds(j * _LANES, _LANES)
            rows = [rows_v[e, sl] for e in range(_N_EXP)]
            m1 = rows[0]
            for e in range(1, _N_EXP):
                m1 = jnp.maximum(m1, rows[e])
            i1 = jnp.full((_LANES,), _N_EXP - 1, dtype=jnp.int32)
            for e in range(_N_EXP - 2, -1, -1):
                i1 = jnp.where(rows[e] == m1, e, i1)
            neg = jnp.float32(-3.0e38)
            rows2 = [jnp.where(i1 == e, neg, rows[e]) for e in range(_N_EXP)]
            m2 = rows2[0]
            for e in range(1, _N_EXP):
                m2 = jnp.maximum(m2, rows2[e])
            i2 = jnp.full((_LANES,), _N_EXP - 1, dtype=jnp.int32)
            for e in range(_N_EXP - 2, -1, -1):
                i2 = jnp.where(rows2[e] == m2, e, i2)
            d = jnp.exp(m2 - m1)
            r = 1.0 / (1.0 + d)
            idx_v[0, sl] = i1
            idx_v[1, sl] = i2
            w_v[0, sl] = r
            w_v[1, sl] = d * r
            return carry

        lax.fori_loop(0, chunk // _LANES, body, 0)
        pltpu.sync_copy(idx_v, idx_hbm.at[:, pl.ds(base, chunk)])
        pltpu.sync_copy(w_v, w_hbm.at[:, pl.ds(base, chunk)])

    return sc_route


@jax.jit
def kernel(hidden_states, weight):
    bsz, seq_len, h = hidden_states.shape
    n_tok = bsz * seq_len
    x = hidden_states.reshape(n_tok, h).astype(jnp.float32)
    wt = weight.astype(jnp.float32).T

    lt = _tc_logits(x, wt)
    idx_t, w_t = _make_sc_route(n_tok)(lt)
    return (idx_t.T, w_t.T)
